# baseline (speedup 1.0000x reference)
import functools

import jax
import jax.numpy as jnp
from jax import lax
from jax.experimental import pallas as pl
from jax.experimental.pallas import tpu as pltpu
from jax.experimental.pallas import tpu_sc as plsc

B = 16384
D = 64
K = 30
KP = 32
TEMPERATURE_EPS = 1e-8

NC, NS, L = 2, 16, 16
NW = NC * NS
RW = B // NW
CR = 4
CI = CR * KP
NCH = RW // CR
NQ = D // L

RBLK = 1024


def _norm_body(zi_ref, zj_ref, pool_ref, pos_ref):
    zi = zi_ref[...]
    zj = zj_ref[...]
    ni = jnp.sqrt(jnp.sum(zi * zi, axis=1, keepdims=True))
    nj = jnp.sqrt(jnp.sum(zj * zj, axis=1, keepdims=True))
    zin = zi / jnp.maximum(ni, TEMPERATURE_EPS)
    zjn = zj / jnp.maximum(nj, TEMPERATURE_EPS)
    pool_ref[0] = zin
    pool_ref[1] = zjn
    pos_ref[...] = jnp.sum(zin * zjn, axis=1, keepdims=True)


_norm_call = pl.pallas_call(
    _norm_body,
    grid=(B // RBLK,),
    in_specs=[
        pl.BlockSpec((RBLK, D), lambda i: (i, 0)),
        pl.BlockSpec((RBLK, D), lambda i: (i, 0)),
    ],
    out_specs=[
        pl.BlockSpec((2, RBLK, D), lambda i: (0, i, 0)),
        pl.BlockSpec((RBLK, 1), lambda i: (i, 0)),
    ],
    out_shape=[
        jax.ShapeDtypeStruct((2, B, D), jnp.float32),
        jax.ShapeDtypeStruct((B, 1), jnp.float32),
    ],
)


_sc_mesh = plsc.VectorSubcoreMesh(
    core_axis_name="c", subcore_axis_name="s", num_cores=NC, num_subcores=NS)


@functools.partial(
    pl.kernel,
    out_type=jax.ShapeDtypeStruct((B, KP), jnp.float32),
    mesh=_sc_mesh,
    compiler_params=pltpu.CompilerParams(
        needs_layout_passes=False, use_tc_tiling_on_sc=False),
    scratch_types=[
        pltpu.VMEM((NCH, CI), jnp.int32),
        pltpu.VMEM((RW, D), jnp.float32),
        pltpu.VMEM((CI, D), jnp.float32),
        pltpu.VMEM((CI, D), jnp.float32),
        pltpu.VMEM((CI, D), jnp.float32),
        pltpu.VMEM((CI, D), jnp.float32),
        pltpu.VMEM((RW, KP), jnp.float32),
        pltpu.VMEM((KP * L,), jnp.float32),
        pltpu.SemaphoreType.DMA,
        pltpu.SemaphoreType.DMA,
        pltpu.SemaphoreType.DMA,
        pltpu.SemaphoreType.DMA,
    ],
)
def _neg_sim_call(pool_hbm, idx_hbm, out_hbm,
                  idx_v, zv, g0, g1, g2, g3, out_v, pacc, s0, s1, s2, s3):
    w = lax.axis_index("s") * NC + lax.axis_index("c")
    base = w * RW
    G = [g0, g1, g2, g3]
    S = [s0, s1, s2, s3]
    RING = 4

    pltpu.sync_copy(idx_hbm.at[w], idx_v)
    pltpu.sync_copy(pool_hbm.at[pl.ds(base, RW)], zv)

    def issue(b, gbuf, sem):
        pltpu.make_async_copy(pool_hbm.at[idx_v.at[b]], gbuf, sem).start()

    def drain(gbuf, sem):
        pltpu.make_async_copy(pool_hbm.at[idx_v.at[0]], gbuf, sem).wait()

    def compute_chunk(b, gbuf):
        row0 = lax.iota(jnp.int32, L) * L

        def row_body(c4, carry):
            r = b * CR + c4
            zrow = zv.at[r]
            za = [zrow[pl.ds(q * L, L)] for q in range(NQ)]
            for k in range(KP):
                grow = gbuf.at[c4 * KP + k]
                acc = za[0] * grow[pl.ds(0, L)]
                for q in range(1, NQ):
                    acc = acc + za[q] * grow[pl.ds(q * L, L)]
                pacc[pl.ds(k * L, L)] = acc
            orow = out_v.at[r]
            for h in range(KP // L):
                rowi = row0 + (h * L * L)
                s = plsc.load_gather(pacc, [rowi])
                for j in range(1, L):
                    s = s + plsc.load_gather(pacc, [rowi + j])
                orow[pl.ds(h * L, L)] = s
            return carry

        lax.fori_loop(0, CR, row_body, 0)

    for p in range(RING):
        issue(p, G[p], S[p])

    def body(jj, carry):
        for p in range(RING):
            b = jj * RING + p
            drain(G[p], S[p])
            compute_chunk(b, G[p])
            issue(jnp.minimum(b + RING, NCH - 1), G[p], S[p])
        return carry

    lax.fori_loop(0, NCH // RING, body, 0)
    for p in range(RING):
        drain(G[p], S[p])
    pltpu.sync_copy(out_v, out_hbm.at[pl.ds(base, RW)])


def _loss_body(t_ref, pos_ref, neg_ref, out_ref):
    i = pl.program_id(0)
    inv_t = 1.0 / t_ref[0, 0]
    pos = pos_ref[...] * inv_t
    neg = neg_ref[...] * inv_t
    col = lax.broadcasted_iota(jnp.int32, neg.shape, 1)
    neg = jnp.where(col < K, neg, -1e30)
    m = jnp.maximum(jnp.max(neg, axis=1, keepdims=True), pos)
    s = jnp.sum(jnp.exp(neg - m), axis=1, keepdims=True) + jnp.exp(pos - m)
    logz = jnp.log(s) + m
    contrib = jnp.sum(logz - pos) * (1.0 / B)

    @pl.when(i == 0)
    def _():
        out_ref[0, 0] = 0.0

    out_ref[0, 0] += contrib


_loss_call = pl.pallas_call(
    _loss_body,
    grid=(B // RBLK,),
    in_specs=[
        pl.BlockSpec(memory_space=pltpu.SMEM),
        pl.BlockSpec((RBLK, 1), lambda i: (i, 0)),
        pl.BlockSpec((RBLK, KP), lambda i: (i, 0)),
    ],
    out_specs=pl.BlockSpec(memory_space=pltpu.SMEM),
    out_shape=jax.ShapeDtypeStruct((1, 1), jnp.float32),
)


def _neg_indices():
    raw = jax.random.randint(jax.random.key(1), (B, K), 0, 2 * B - 2)
    i = jnp.arange(B, dtype=raw.dtype)[:, None]
    r = raw + (raw >= i).astype(raw.dtype)
    r = r + (r >= (i + B)).astype(raw.dtype)
    return r


def kernel(z_i, z_j, temperature):
    pool, pos = _norm_call(z_i, z_j)
    pool2 = pool.reshape(2 * B, D)
    idx = _neg_indices()
    idx_p = jnp.concatenate(
        [idx, jnp.zeros((B, KP - K), idx.dtype)], axis=1)
    idx_sc = idx_p.reshape(NW, NCH, CI)
    neg = _neg_sim_call(pool2, idx_sc)
    t = jnp.asarray(temperature, jnp.float32).reshape(1, 1)
    loss = _loss_call(t, pos, neg)
    return loss[0, 0]


# speedup vs baseline: 1.7174x; 1.7174x over previous
import functools

import jax
import jax.numpy as jnp
from jax import lax
from jax.experimental import pallas as pl
from jax.experimental.pallas import tpu as pltpu
from jax.experimental.pallas import tpu_sc as plsc

B = 16384
D = 64
DP = D // 2
K = 30
KP = 32
TEMPERATURE_EPS = 1e-8

NC, NS, L = 2, 16, 16
NW = NC * NS
RW = B // NW
CR = 4
CI = CR * KP
NCH = RW // CR
NQ = DP // L

RBLK = 1024


def _pack_rows(x):
    xb = x.astype(jnp.bfloat16)
    lo = lax.bitcast_convert_type(xb[:, :DP], jnp.uint16).astype(jnp.uint32)
    hi = lax.bitcast_convert_type(xb[:, DP:], jnp.uint16).astype(jnp.uint32)
    return lax.bitcast_convert_type(lo | (hi << 16), jnp.float32)


def _norm_body(zi_ref, zj_ref, pool_ref, pos_ref):
    zi = zi_ref[...]
    zj = zj_ref[...]
    ni = jnp.sqrt(jnp.sum(zi * zi, axis=1, keepdims=True))
    nj = jnp.sqrt(jnp.sum(zj * zj, axis=1, keepdims=True))
    zin = zi / jnp.maximum(ni, TEMPERATURE_EPS)
    zjn = zj / jnp.maximum(nj, TEMPERATURE_EPS)
    pool_ref[0] = _pack_rows(zin)
    pool_ref[1] = _pack_rows(zjn)
    pos_ref[...] = jnp.sum(zin * zjn, axis=1, keepdims=True)


_norm_call = pl.pallas_call(
    _norm_body,
    grid=(B // RBLK,),
    in_specs=[
        pl.BlockSpec((RBLK, D), lambda i: (i, 0)),
        pl.BlockSpec((RBLK, D), lambda i: (i, 0)),
    ],
    out_specs=[
        pl.BlockSpec((2, RBLK, DP), lambda i: (0, i, 0)),
        pl.BlockSpec((RBLK, 1), lambda i: (i, 0)),
    ],
    out_shape=[
        jax.ShapeDtypeStruct((2, B, DP), jnp.float32),
        jax.ShapeDtypeStruct((B, 1), jnp.float32),
    ],
)


_sc_mesh = plsc.VectorSubcoreMesh(
    core_axis_name="c", subcore_axis_name="s", num_cores=NC, num_subcores=NS)


@functools.partial(
    pl.kernel,
    out_type=jax.ShapeDtypeStruct((B, KP), jnp.float32),
    mesh=_sc_mesh,
    compiler_params=pltpu.CompilerParams(
        needs_layout_passes=False, use_tc_tiling_on_sc=False),
    scratch_types=[
        pltpu.VMEM((NCH, CI), jnp.int32),
        pltpu.VMEM((RW, DP), jnp.float32),
        pltpu.VMEM((CI, DP), jnp.float32),
        pltpu.VMEM((CI, DP), jnp.float32),
        pltpu.VMEM((CI, DP), jnp.float32),
        pltpu.VMEM((CI, DP), jnp.float32),
        pltpu.VMEM((RW, KP), jnp.float32),
        pltpu.VMEM((KP * L,), jnp.float32),
        pltpu.SemaphoreType.DMA,
        pltpu.SemaphoreType.DMA,
        pltpu.SemaphoreType.DMA,
        pltpu.SemaphoreType.DMA,
    ],
)
def _neg_sim_call(pool_hbm, idx_hbm, out_hbm,
                  idx_v, zv, g0, g1, g2, g3, out_v, pacc, s0, s1, s2, s3):
    w = lax.axis_index("s") * NC + lax.axis_index("c")
    base = w * RW
    G = [g0, g1, g2, g3]
    S = [s0, s1, s2, s3]
    RING = 4

    pltpu.sync_copy(idx_hbm.at[w], idx_v)
    pltpu.sync_copy(pool_hbm.at[pl.ds(base, RW)], zv)

    def issue(b, gbuf, sem):
        pltpu.make_async_copy(pool_hbm.at[idx_v.at[b]], gbuf, sem).start()

    def drain(gbuf, sem):
        pltpu.make_async_copy(pool_hbm.at[idx_v.at[0]], gbuf, sem).wait()

    def compute_chunk(b, gbuf):
        row0 = lax.iota(jnp.int32, L) * L

        def row_body(c4, carry):
            r = b * CR + c4
            zrow = zv.at[r]
            za = [plsc.bitcast(zrow[pl.ds(q * L, L)], jnp.bfloat16)
                  for q in range(NQ)]
            for k in range(KP):
                grow = gbuf.at[c4 * KP + k]
                acc = None
                for q in range(NQ):
                    gq = plsc.bitcast(grow[pl.ds(q * L, L)], jnp.bfloat16)
                    p0, p1 = plsc.unpack(za[q] * gq,
                                         format=plsc.PackFormat.INTERLEAVED)
                    ps = p0 + p1
                    acc = ps if acc is None else acc + ps
                pacc[pl.ds(k * L, L)] = acc
            orow = out_v.at[r]
            for h in range(KP // L):
                rowi = row0 + (h * L * L)
                s = plsc.load_gather(pacc, [rowi])
                for j in range(1, L):
                    s = s + plsc.load_gather(pacc, [rowi + j])
                orow[pl.ds(h * L, L)] = s
            return carry

        lax.fori_loop(0, CR, row_body, 0)

    for p in range(RING):
        issue(p, G[p], S[p])

    def body(jj, carry):
        for p in range(RING):
            b = jj * RING + p
            drain(G[p], S[p])
            compute_chunk(b, G[p])
            issue(jnp.minimum(b + RING, NCH - 1), G[p], S[p])
        return carry

    lax.fori_loop(0, NCH // RING, body, 0)
    for p in range(RING):
        drain(G[p], S[p])
    pltpu.sync_copy(out_v, out_hbm.at[pl.ds(base, RW)])


def _loss_body(t_ref, pos_ref, neg_ref, out_ref):
    i = pl.program_id(0)
    inv_t = 1.0 / t_ref[0, 0]
    pos = pos_ref[...] * inv_t
    neg = neg_ref[...] * inv_t
    col = lax.broadcasted_iota(jnp.int32, neg.shape, 1)
    neg = jnp.where(col < K, neg, -1e30)
    m = jnp.maximum(jnp.max(neg, axis=1, keepdims=True), pos)
    s = jnp.sum(jnp.exp(neg - m), axis=1, keepdims=True) + jnp.exp(pos - m)
    logz = jnp.log(s) + m
    contrib = jnp.sum(logz - pos) * (1.0 / B)

    @pl.when(i == 0)
    def _():
        out_ref[0, 0] = 0.0

    out_ref[0, 0] += contrib


_loss_call = pl.pallas_call(
    _loss_body,
    grid=(B // RBLK,),
    in_specs=[
        pl.BlockSpec(memory_space=pltpu.SMEM),
        pl.BlockSpec((RBLK, 1), lambda i: (i, 0)),
        pl.BlockSpec((RBLK, KP), lambda i: (i, 0)),
    ],
    out_specs=pl.BlockSpec(memory_space=pltpu.SMEM),
    out_shape=jax.ShapeDtypeStruct((1, 1), jnp.float32),
)


def _neg_indices():
    raw = jax.random.randint(jax.random.key(1), (B, K), 0, 2 * B - 2)
    i = jnp.arange(B, dtype=raw.dtype)[:, None]
    r = raw + (raw >= i).astype(raw.dtype)
    r = r + (r >= (i + B)).astype(raw.dtype)
    return r


def kernel(z_i, z_j, temperature):
    pool, pos = _norm_call(z_i, z_j)
    pool2 = pool.reshape(2 * B, DP)
    idx = _neg_indices()
    idx_p = jnp.concatenate(
        [idx, jnp.zeros((B, KP - K), idx.dtype)], axis=1)
    idx_sc = idx_p.reshape(NW, NCH, CI)
    neg = _neg_sim_call(pool2, idx_sc)
    t = jnp.asarray(temperature, jnp.float32).reshape(1, 1)
    loss = _loss_call(t, pos, neg)
    return loss[0, 0]
